# baseline (speedup 1.0000x reference)
import functools

import jax
import jax.numpy as jnp
from jax import lax
from jax.experimental import pallas as pl
from jax.experimental.pallas import tpu as pltpu
from jax.experimental.pallas import tpu_sc as plsc

_NC = 2
_NS = 16
_LANES = 16
_PAD = 64
_SC_B0 = 8
_WPB = 4


def _sc_tail_sum(hidden, mask, xs):
    B, L, D = hidden.shape
    nb = B - _SC_B0
    Dc = D // _WPB
    nchunk = Dc // _LANES
    mesh = plsc.VectorSubcoreMesh(core_axis_name="c", subcore_axis_name="s")

    @functools.partial(
        pl.kernel,
        out_type=jax.ShapeDtypeStruct((nb, 1, D), jnp.float32),
        mesh=mesh,
        compiler_params=pltpu.CompilerParams(
            needs_layout_passes=False,
            skip_device_barrier=True,
        ),
        scratch_types=[
            pltpu.VMEM((_LANES,), jnp.int32),
            pltpu.VMEM((1, L), jnp.int32),
            pltpu.VMEM((_LANES,), jnp.int32),
            pltpu.VMEM((_PAD, Dc), jnp.float32),
            pltpu.VMEM((Dc,), jnp.float32),
            pltpu.SemaphoreType.DMA,
            pltpu.SemaphoreType.DMA,
            pltpu.SemaphoreType.DMA,
        ],
    )
    def k(hidden_hbm, mask_hbm, xs_hbm, out_hbm,
          idx1_v, mask_v, xs_v, win_v, acc_v, semw, semm, semx):
        c = lax.axis_index("c")
        s = lax.axis_index("s")
        wid = s * _NC + c
        bo = wid // _WPB
        b = _SC_B0 + bo
        dcol = (wid % _WPB) * Dc

        wcopy = pltpu.async_copy(
            hidden_hbm.at[b, pl.ds(L - _PAD, _PAD), pl.ds(dcol, Dc)],
            win_v, semw,
        )

        idx1_v[...] = jnp.zeros((_LANES,), jnp.int32) + b
        mcopy = pltpu.async_copy(
            mask_hbm.at[idx1_v.at[pl.ds(0, 1)]], mask_v, semm
        )
        xcopy = pltpu.async_copy(xs_hbm, xs_v, semx)
        mcopy.wait()

        accs = [jnp.zeros((_LANES,), jnp.int32) for _ in range(4)]
        for i in range(L // (_LANES * 4)):
            for u in range(4):
                accs[u] = accs[u] + mask_v[0, pl.ds((4 * i + u) * _LANES, _LANES)]
        hi = jnp.sum(accs[0] + accs[1] + (accs[2] + accs[3]))
        xcopy.wait()
        x_s = jnp.max(xs_v[...])

        lo = jnp.maximum(hi - x_s, 0)
        base = jnp.minimum((lo // 8) * 8, L - _PAD)

        wcopy.wait()

        @pl.when(base != L - _PAD)
        def _respin():
            pltpu.sync_copy(
                hidden_hbm.at[b, pl.ds(base, _PAD), pl.ds(dcol, Dc)], win_v
            )

        lo_idx = lo - base
        hi_idx = hi - base
        nrows = hi_idx - lo_idx
        npairs = nrows // 2

        def row2(jj, acc):
            j0 = lo_idx + 2 * jj
            return tuple(
                acc[t]
                + win_v[j0, pl.ds(t * _LANES, _LANES)]
                + win_v[j0 + 1, pl.ds(t * _LANES, _LANES)]
                for t in range(nchunk)
            )

        acc0 = tuple(jnp.zeros((_LANES,), jnp.float32) for _ in range(nchunk))
        acc = lax.fori_loop(0, npairs, row2, acc0)
        jlast = jnp.minimum(lo_idx + 2 * npairs, _PAD - 1)
        wodd = (nrows - 2 * npairs).astype(jnp.float32)
        acc = tuple(
            acc[t] + wodd * win_v[jlast, pl.ds(t * _LANES, _LANES)]
            for t in range(nchunk)
        )
        for t in range(nchunk):
            acc_v[pl.ds(t * _LANES, _LANES)] = acc[t]
        pltpu.sync_copy(acc_v, out_hbm.at[bo, 0, pl.ds(dcol, Dc)])

    return k(hidden, mask, xs)


def _tc_tail_sum(hidden, mask, xs):
    B, L, D = hidden.shape
    nb = _SC_B0

    def body(mask_ref, xs_ref, hidden_any, out_ref, win_v, sem):
        g = pl.program_id(0)
        hi = jnp.sum(mask_ref[0])
        x_s = xs_ref[0]
        lo = jnp.maximum(hi - x_s, 0)
        base = jnp.minimum((lo // 8) * 8, L - _PAD)
        pltpu.make_async_copy(
            hidden_any.at[g, pl.ds(base, _PAD), :], win_v, sem
        ).start()
        rows = lax.broadcasted_iota(jnp.int32, (_PAD, 1), 0) + base
        w = ((rows >= lo) & (rows < hi)).astype(jnp.float32)
        pltpu.make_async_copy(
            hidden_any.at[g, pl.ds(base, _PAD), :], win_v, sem
        ).wait()
        out_ref[...] = jnp.sum(win_v[...] * w, axis=0, keepdims=True)[None]

    return pl.pallas_call(
        body,
        grid=(nb,),
        in_specs=[
            pl.BlockSpec((1, 1, L), lambda g: (g, 0, 0)),
            pl.BlockSpec(memory_space=pltpu.SMEM),
            pl.BlockSpec(memory_space=pl.ANY),
        ],
        out_specs=pl.BlockSpec((1, 1, D), lambda g: (g, 0, 0)),
        out_shape=jax.ShapeDtypeStruct((nb, 1, D), jnp.float32),
        scratch_shapes=[
            pltpu.VMEM((_PAD, D), jnp.float32),
            pltpu.SemaphoreType.DMA,
        ],
    )(mask[:nb].reshape(nb, 1, L), xs, hidden)


def kernel(hidden, mask, x):
    B, L, D = hidden.shape
    assert B == _NS and D % (_WPB * _LANES) == 0 and L % (_LANES * 4) == 0
    assert L >= _PAD and L % 8 == 0 and (B - _SC_B0) * _WPB == _NC * _NS
    xs = jnp.full((_LANES,), x, dtype=jnp.int32)
    mask_i = mask.astype(jnp.int32)
    out_sc = _sc_tail_sum(hidden, mask_i, xs)
    out_tc = _tc_tail_sum(hidden, mask_i, xs)
    out = jnp.concatenate([out_tc, out_sc], axis=0)
    return out.astype(hidden.dtype)


# speedup vs baseline: 1.2571x; 1.2571x over previous
import functools

import jax
import jax.numpy as jnp
from jax import lax
from jax.experimental import pallas as pl
from jax.experimental.pallas import tpu as pltpu
from jax.experimental.pallas import tpu_sc as plsc

_NC = 2
_NS = 16
_LANES = 16
_PAD = 64
_NQ = 4
_QR = _PAD // _NQ


def _sc_tail_sum(hidden, mask, xs):
    B, L, D = hidden.shape
    Dc = D // _NC
    nchunk = Dc // _LANES
    mesh = plsc.VectorSubcoreMesh(core_axis_name="c", subcore_axis_name="s")

    @functools.partial(
        pl.kernel,
        out_type=jax.ShapeDtypeStruct((B, 1, D), jnp.float32),
        mesh=mesh,
        compiler_params=pltpu.CompilerParams(
            needs_layout_passes=False,
            skip_device_barrier=True,
        ),
        scratch_types=[
            pltpu.VMEM((_LANES,), jnp.int32),
            pltpu.VMEM((1, L), jnp.int32),
            pltpu.VMEM((_LANES,), jnp.int32),
            pltpu.VMEM((_PAD, Dc), jnp.float32),
            pltpu.VMEM((Dc,), jnp.float32),
            pltpu.SemaphoreType.DMA,
            pltpu.SemaphoreType.DMA,
            pltpu.SemaphoreType.DMA,
            pltpu.SemaphoreType.DMA,
            pltpu.SemaphoreType.DMA,
            pltpu.SemaphoreType.DMA,
        ],
    )
    def k(hidden_hbm, mask_hbm, xs_hbm, out_hbm,
          idx1_v, mask_v, xs_v, win_v, acc_v,
          semq0, semq1, semq2, semq3, semm, semx):
        c = lax.axis_index("c")
        s = lax.axis_index("s")
        b = s
        dcol = c * Dc
        semq = [semq0, semq1, semq2, semq3]

        qcopies = [
            pltpu.async_copy(
                hidden_hbm.at[b, pl.ds(L - _PAD + q * _QR, _QR),
                              pl.ds(dcol, Dc)],
                win_v.at[pl.ds(q * _QR, _QR)], semq[q],
            )
            for q in range(_NQ)
        ]

        idx1_v[...] = jnp.zeros((_LANES,), jnp.int32) + b
        mcopy = pltpu.async_copy(
            mask_hbm.at[idx1_v.at[pl.ds(0, 1)]], mask_v, semm
        )
        xcopy = pltpu.async_copy(xs_hbm, xs_v, semx)
        mcopy.wait()

        accs = [jnp.zeros((_LANES,), jnp.int32) for _ in range(4)]
        for i in range(L // (_LANES * 4)):
            for u in range(4):
                accs[u] = accs[u] + mask_v[0, pl.ds((4 * i + u) * _LANES, _LANES)]
        hi = jnp.sum(accs[0] + accs[1] + (accs[2] + accs[3]))
        xcopy.wait()
        x_s = jnp.max(xs_v[...])

        lo = jnp.maximum(hi - x_s, 0)
        base = jnp.minimum((lo // 8) * 8, L - _PAD)
        lo_idx = lo - base
        hi_idx = hi - base

        def row(j, acc):
            return tuple(
                acc[t] + win_v[j, pl.ds(t * _LANES, _LANES)]
                for t in range(nchunk)
            )

        acc0 = tuple(jnp.zeros((_LANES,), jnp.float32) for _ in range(nchunk))

        @pl.when(base == L - _PAD)
        def _hit():
            acc = acc0
            for q in range(_NQ):
                qcopies[q].wait()
                ql = jnp.maximum(lo_idx, q * _QR)
                qh = jnp.minimum(hi_idx, (q + 1) * _QR)
                acc = lax.fori_loop(ql, qh, row, acc)
            for t in range(nchunk):
                acc_v[pl.ds(t * _LANES, _LANES)] = acc[t]

        @pl.when(base != L - _PAD)
        def _miss():
            for q in range(_NQ):
                qcopies[q].wait()
            pltpu.sync_copy(
                hidden_hbm.at[b, pl.ds(base, _PAD), pl.ds(dcol, Dc)], win_v
            )
            acc = lax.fori_loop(lo_idx, hi_idx, row, acc0)
            for t in range(nchunk):
                acc_v[pl.ds(t * _LANES, _LANES)] = acc[t]

        pltpu.sync_copy(acc_v, out_hbm.at[b, 0, pl.ds(dcol, Dc)])

    return k(hidden, mask, xs)


def kernel(hidden, mask, x):
    B, L, D = hidden.shape
    assert B == _NS and D % (_NC * _LANES) == 0 and L % (_LANES * 4) == 0
    assert L >= _PAD and L % 8 == 0
    xs = jnp.full((_LANES,), x, dtype=jnp.int32)
    out = _sc_tail_sum(hidden, mask.astype(jnp.int32), xs)
    return out.astype(hidden.dtype)


# speedup vs baseline: 1.3381x; 1.0645x over previous
import functools

import jax
import jax.numpy as jnp
from jax import lax
from jax.experimental import pallas as pl
from jax.experimental.pallas import tpu as pltpu
from jax.experimental.pallas import tpu_sc as plsc

_NC = 2
_NS = 16
_LANES = 16
_PAD = 64


def _sc_tail_sum(hidden, mask, xs):
    B, L, D = hidden.shape
    Dc = D // _NC
    nchunk = Dc // _LANES
    mesh = plsc.VectorSubcoreMesh(core_axis_name="c", subcore_axis_name="s")

    @functools.partial(
        pl.kernel,
        out_type=jax.ShapeDtypeStruct((B, 1, D), jnp.float32),
        mesh=mesh,
        compiler_params=pltpu.CompilerParams(
            needs_layout_passes=False,
            skip_device_barrier=True,
        ),
        scratch_types=[
            pltpu.VMEM((_LANES,), jnp.int32),
            pltpu.VMEM((1, L), jnp.int32),
            pltpu.VMEM((_LANES,), jnp.int32),
            pltpu.VMEM((_PAD, Dc), jnp.float32),
            pltpu.VMEM((Dc,), jnp.float32),
            pltpu.SemaphoreType.DMA,
            pltpu.SemaphoreType.DMA,
            pltpu.SemaphoreType.DMA,
        ],
    )
    def k(hidden_hbm, mask_hbm, xs_hbm, out_hbm,
          idx1_v, mask_v, xs_v, win_v, acc_v, semw, semm, semx):
        c = lax.axis_index("c")
        s = lax.axis_index("s")
        b = s
        dcol = c * Dc

        wcopy = pltpu.async_copy(
            hidden_hbm.at[b, pl.ds(L - _PAD, _PAD), pl.ds(dcol, Dc)],
            win_v, semw,
        )

        idx1_v[...] = jnp.zeros((_LANES,), jnp.int32) + b
        mcopy = pltpu.async_copy(
            mask_hbm.at[idx1_v.at[pl.ds(0, 1)]], mask_v, semm
        )
        xcopy = pltpu.async_copy(xs_hbm, xs_v, semx)
        mcopy.wait()

        accs = [jnp.zeros((_LANES,), jnp.int32) for _ in range(4)]
        for i in range(L // (_LANES * 4)):
            for u in range(4):
                accs[u] = accs[u] + mask_v[0, pl.ds((4 * i + u) * _LANES, _LANES)]
        hi = jnp.sum(accs[0] + accs[1] + (accs[2] + accs[3]))
        xcopy.wait()
        x_s = jnp.max(xs_v[...])

        lo = jnp.maximum(hi - x_s, 0)
        base = jnp.minimum((lo // 8) * 8, L - _PAD)

        wcopy.wait()

        @pl.when(base != L - _PAD)
        def _respin():
            pltpu.sync_copy(
                hidden_hbm.at[b, pl.ds(base, _PAD), pl.ds(dcol, Dc)], win_v
            )

        lo_idx = lo - base
        hi_idx = hi - base

        def row(j, acc):
            return tuple(
                acc[t] + win_v[j, pl.ds(t * _LANES, _LANES)]
                for t in range(nchunk)
            )

        acc0 = tuple(jnp.zeros((_LANES,), jnp.float32) for _ in range(nchunk))
        acc = lax.fori_loop(lo_idx, hi_idx, row, acc0)
        for t in range(nchunk):
            acc_v[pl.ds(t * _LANES, _LANES)] = acc[t]
        pltpu.sync_copy(acc_v, out_hbm.at[b, 0, pl.ds(dcol, Dc)])

    return k(hidden, mask, xs)


def kernel(hidden, mask, x):
    B, L, D = hidden.shape
    assert B == _NS and D % (_NC * _LANES) == 0 and L % (_LANES * 4) == 0
    assert L >= _PAD and L % 8 == 0
    xs = jnp.full((_LANES,), x, dtype=jnp.int32)
    out = _sc_tail_sum(hidden, mask.astype(jnp.int32), xs)
    return out.astype(hidden.dtype)
